# Initial kernel scaffold; baseline (speedup 1.0000x reference)
import jax
import jax.numpy as jnp
from jax.experimental import pallas as pl


def kernel(x, W_gate, w1, w3, w2):
    raise NotImplementedError("write your pallas kernel here")


# speedup vs baseline: 1.1517x; 1.1517x over previous
import functools

import jax
import jax.numpy as jnp
from jax import lax
from jax.experimental import pallas as pl
from jax.experimental.pallas import tpu as pltpu

K = 2
BR = 512
BM = 256
BF = 512

_INTERPRET = False


def _router_body(x_ref, wg_ref, e0_ref, e1_ref, w0_ref, w1_ref, stats_ref):
    E = wg_ref.shape[-1]
    x = x_ref[...]
    logits = jnp.dot(x, wg_ref[...], preferred_element_type=jnp.float32)
    m = jnp.max(logits, axis=-1, keepdims=True)
    ex = jnp.exp(logits - m)
    s = jnp.sum(ex, axis=-1, keepdims=True)
    probs = ex / s
    lse = m[:, 0] + jnp.log(s[:, 0])

    eidx = lax.broadcasted_iota(jnp.int32, probs.shape, 1)
    p0 = jnp.max(probs, axis=-1)
    is0 = probs == p0[:, None]
    i0 = jnp.min(jnp.where(is0, eidx, E), axis=-1).astype(jnp.int32)
    probs_m = jnp.where(eidx == i0[:, None], -1.0, probs)
    p1 = jnp.max(probs_m, axis=-1)
    is1 = probs_m == p1[:, None]
    i1 = jnp.min(jnp.where(is1, eidx, E), axis=-1).astype(jnp.int32)
    tot = p0 + p1
    e0_ref[...] = i0
    e1_ref[...] = i1
    w0_ref[...] = p0 / tot
    w1_ref[...] = p1 / tot

    onehot = (eidx == i0[:, None]).astype(jnp.float32) + (
        eidx == i1[:, None]
    ).astype(jnp.float32)
    psum = jnp.sum(probs, axis=0)
    csum = jnp.sum(onehot, axis=0)
    zsum = jnp.sum(lse * lse)
    vec = jnp.concatenate(
        [psum, csum, jnp.full((1,), zsum, jnp.float32),
         jnp.zeros((128 - 2 * E - 1,), jnp.float32)]
    )
    stats_ref[0, 0, :] = vec


def _router(xf, wgT):
    T, H = xf.shape
    E = wgT.shape[1]
    nblk = T // BR
    return pl.pallas_call(
        _router_body,
        grid=(nblk,),
        in_specs=[
            pl.BlockSpec((BR, H), lambda i: (i, 0)),
            pl.BlockSpec((H, E), lambda i: (0, 0)),
        ],
        out_specs=[
            pl.BlockSpec((BR,), lambda i: (i,)),
            pl.BlockSpec((BR,), lambda i: (i,)),
            pl.BlockSpec((BR,), lambda i: (i,)),
            pl.BlockSpec((BR,), lambda i: (i,)),
            pl.BlockSpec((1, 1, 128), lambda i: (i, 0, 0)),
        ],
        out_shape=[
            jax.ShapeDtypeStruct((T,), jnp.int32),
            jax.ShapeDtypeStruct((T,), jnp.int32),
            jax.ShapeDtypeStruct((T,), jnp.float32),
            jax.ShapeDtypeStruct((T,), jnp.float32),
            jax.ShapeDtypeStruct((nblk, 1, 128), jnp.float32),
        ],
        interpret=_INTERPRET,
    )(xf, wgT)


def _ffn_body(be_ref, x_ref, w1_ref, w3_ref, w2_ref, wt_ref, y_ref, acc_ref):
    nj = pl.num_programs(1)
    j = pl.program_id(1)
    x = x_ref[...]
    a = jnp.dot(x, w1_ref[0], preferred_element_type=jnp.float32)
    b = jnp.dot(x, w3_ref[0], preferred_element_type=jnp.float32)
    h = a * jax.nn.sigmoid(a) * b
    part = jnp.dot(h, w2_ref[0], preferred_element_type=jnp.float32)

    @pl.when(j == 0)
    def _():
        acc_ref[...] = part

    @pl.when(j > 0)
    def _():
        acc_ref[...] += part

    @pl.when(j == nj - 1)
    def _():
        y_ref[...] = acc_ref[...] * wt_ref[...][:, None]


def _ffn(block_expert, x_pad, w1, w3, w2, wt_pad):
    PT, H = x_pad.shape
    E, _, F = w1.shape
    nb, nj = PT // BM, F // BF
    grid_spec = pltpu.PrefetchScalarGridSpec(
        num_scalar_prefetch=1,
        grid=(nb, nj),
        in_specs=[
            pl.BlockSpec((BM, H), lambda i, j, be: (i, 0)),
            pl.BlockSpec((1, H, BF), lambda i, j, be: (be[i], 0, j)),
            pl.BlockSpec((1, H, BF), lambda i, j, be: (be[i], 0, j)),
            pl.BlockSpec((1, BF, H), lambda i, j, be: (be[i], j, 0)),
            pl.BlockSpec((BM,), lambda i, j, be: (i,)),
        ],
        out_specs=pl.BlockSpec((BM, H), lambda i, j, be: (i, 0)),
        scratch_shapes=[pltpu.VMEM((BM, H), jnp.float32)],
    )
    return pl.pallas_call(
        _ffn_body,
        grid_spec=grid_spec,
        out_shape=jax.ShapeDtypeStruct((PT, H), jnp.float32),
        compiler_params=pltpu.CompilerParams(
            dimension_semantics=("arbitrary", "arbitrary"),
        ),
        interpret=_INTERPRET,
    )(block_expert, x_pad, w1, w3, w2, wt_pad)


def kernel(x, W_gate, w1, w3, w2):
    b, s, H = x.shape
    T = b * s
    E = W_gate.shape[0]
    xf = x.reshape(T, H)

    e0, e1, w0v, w1v, stats = _router(xf, W_gate.T)
    ssum = jnp.sum(stats, axis=(0, 1))
    probs_sum = ssum[:E]
    counts = ssum[E:2 * E]
    zsum = ssum[2 * E]
    balance_loss = E * jnp.sum((counts / (T * K)) * (probs_sum / T))
    z_loss = zsum / T

    PT = T * K + E * BM
    e_flat = jnp.stack([e0, e1], axis=1).reshape(-1)
    onehot = (e_flat[:, None] == jnp.arange(E)[None, :]).astype(jnp.int32)
    csum = jnp.cumsum(onehot, axis=0)
    rank = jnp.sum((csum - 1) * onehot, axis=1)
    g = csum[-1]
    padded = ((g + BM - 1) // BM) * BM
    pstart = jnp.concatenate(
        [jnp.zeros((1,), jnp.int32), jnp.cumsum(padded)[:-1].astype(jnp.int32)]
    )
    dest = pstart[e_flat] + rank
    tok = jnp.repeat(jnp.arange(T, dtype=jnp.int32), K)
    src_token = jnp.zeros((PT,), jnp.int32).at[dest].set(tok)
    wt_pad = jnp.zeros((PT,), jnp.float32).at[dest].set(
        jnp.stack([w0v, w1v], axis=1).reshape(-1)
    )
    nb = PT // BM
    bstart = jnp.arange(nb, dtype=jnp.int32) * BM
    block_expert = jnp.clip(
        jnp.searchsorted(pstart, bstart, side="right").astype(jnp.int32) - 1,
        0, E - 1,
    )
    d0 = dest[0::K]
    d1 = dest[1::K]

    x_pad = xf[src_token]

    y = _ffn(block_expert, x_pad, w1, w3, w2, wt_pad)

    out = y[d0] + y[d1]

    return out.reshape(b, s, H), balance_loss, z_loss
